# baseline (speedup 1.0000x reference)
import functools

import jax
import jax.numpy as jnp
from jax import lax
from jax.experimental import pallas as pl
from jax.experimental.pallas import tpu as pltpu
from jax.experimental.pallas import tpu_sc as plsc

KNN = 20
KPAD = 24
RT = 256
CHUNK = 128


def _topk_body(cc_ref, ct_ref, out_ref):
    n = cc_ref.shape[1]
    r = ct_ref.shape[2]
    cc = cc_ref[0]
    ct = ct_ref[0]
    iota_f = lax.broadcasted_iota(jnp.int32, (n, r), 0).astype(jnp.float32)
    col_f = (lax.broadcasted_iota(jnp.int32, (n, r), 1)
             + pl.program_id(1) * r).astype(jnp.float32)
    d = ((cc[:, 0:1] - ct[0:1, :]) ** 2
         + (cc[:, 1:2] - ct[1:2, :]) ** 2)
    d = jnp.where(iota_f == col_f, jnp.inf, d)
    base = pl.program_id(0) * n
    for j in range(KNN):
        m = jnp.min(d, axis=0, keepdims=True)
        msk = d == m
        a_f = jnp.min(jnp.where(msk, iota_f, 4096.0), axis=0)
        out_ref[0, j, :] = a_f.astype(jnp.int32) + base
        if j < KNN - 1:
            d = jnp.where(msk, jnp.inf, d)
    out_ref[0, KNN:KPAD, :] = jnp.zeros((KPAD - KNN, r), jnp.int32)


def _mlp_body(x_ref, nb_ref, w1_ref, b1_ref, w2_ref, b2_ref, out_ref):
    d = x_ref.shape[2]
    r = x_ref.shape[1]
    w1a = w1_ref[0:d, :]
    w1b = w1_ref[d:2 * d, :]
    u = jnp.dot(x_ref[0], w1a - w1b,
                preferred_element_type=jnp.float32) + b1_ref[...]
    nb = nb_ref[0].reshape(KNN * r, d)
    g = jnp.dot(nb, w1b.astype(jnp.bfloat16),
                preferred_element_type=jnp.float32)
    units = g.shape[1]
    h1 = jnp.maximum(g.reshape(KNN, r, units) + u[None], 0.0)
    h2 = jnp.maximum(
        jnp.dot(h1.reshape(KNN * r, units).astype(jnp.bfloat16),
                w2_ref[...].astype(jnp.bfloat16),
                preferred_element_type=jnp.float32) + b2_ref[...], 0.0)
    out_ref[0] = jnp.max(h2.reshape(KNN, r, units), axis=0)


def _sc_gather(xf, idx2, b_sz, n, d):
    info = plsc.get_sparse_core_info()
    nc, ns = info.num_cores, info.num_subcores
    nw = nc * ns
    n_tasks = b_sz * KNN
    tasks_per_w = n_tasks // nw
    chunks = n // CHUNK
    mesh = plsc.VectorSubcoreMesh(core_axis_name="c", subcore_axis_name="s")

    @functools.partial(
        pl.kernel, mesh=mesh,
        out_type=jax.ShapeDtypeStruct((b_sz, KNN, n, d), jnp.bfloat16),
        scratch_types=[
            pltpu.VMEM((chunks, CHUNK), jnp.int32),
            pltpu.VMEM((CHUNK, d), jnp.bfloat16),
            pltpu.SemaphoreType.DMA,
        ],
        compiler_params=pltpu.CompilerParams(use_tc_tiling_on_sc=False),
    )
    def gather_kernel(xf_hbm, idx_hbm, out_hbm, idx_v, rows_v, sem):
        wid = lax.axis_index("s") * nc + lax.axis_index("c")
        for t in range(tasks_per_w):
            tau = wid * tasks_per_w + t
            b = tau // KNN
            k = tau - b * KNN
            irow = (b * KPAD + k) * chunks
            pltpu.sync_copy(idx_hbm.at[pl.ds(irow, chunks)], idx_v)

            def chunk_body(c, carry):
                pltpu.async_copy(xf_hbm.at[idx_v.at[c]], rows_v, sem).wait()
                pltpu.sync_copy(rows_v,
                                out_hbm.at[b, k, pl.ds(c * CHUNK, CHUNK)])
                return carry

            lax.fori_loop(0, chunks, chunk_body, 0)

    return gather_kernel(xf, idx2)


def kernel(x, W1, b1, W2, b2):
    B, N, D = x.shape
    units = W2.shape[0]
    coords = x[:, :, 1:3]
    cc = jnp.pad(coords, ((0, 0), (0, 0), (0, 6)))
    ct = jnp.pad(jnp.swapaxes(coords, 1, 2), ((0, 0), (0, 6), (0, 0)))

    idxg = pl.pallas_call(
        _topk_body,
        grid=(B, N // RT),
        in_specs=[
            pl.BlockSpec((1, N, 8), lambda b, t: (b, 0, 0)),
            pl.BlockSpec((1, 8, RT), lambda b, t: (b, 0, t)),
        ],
        out_specs=pl.BlockSpec((1, KPAD, RT), lambda b, t: (b, 0, t)),
        out_shape=jax.ShapeDtypeStruct((B, KPAD, N), jnp.int32),
    )(cc, ct)

    idx2 = idxg.reshape(B * KPAD * (N // CHUNK), CHUNK)
    xf = x.astype(jnp.bfloat16).reshape(B * N, D)
    nb4 = _sc_gather(xf, idx2, B, N, D)

    out = pl.pallas_call(
        _mlp_body,
        grid=(B, N // RT),
        in_specs=[
            pl.BlockSpec((1, RT, D), lambda b, t: (b, t, 0)),
            pl.BlockSpec((1, KNN, RT, D), lambda b, t: (b, 0, t, 0)),
            pl.BlockSpec((2 * D, units), lambda b, t: (0, 0)),
            pl.BlockSpec((1, units), lambda b, t: (0, 0)),
            pl.BlockSpec((units, units), lambda b, t: (0, 0)),
            pl.BlockSpec((1, units), lambda b, t: (0, 0)),
        ],
        out_specs=pl.BlockSpec((1, RT, units), lambda b, t: (b, t, 0)),
        out_shape=jax.ShapeDtypeStruct((B, N, units), jnp.float32),
    )(x, nb4, W1, b1.reshape(1, units), W2, b2.reshape(1, units))
    return out


# speedup vs baseline: 1.0219x; 1.0219x over previous
import functools

import jax
import jax.numpy as jnp
from jax import lax
from jax.experimental import pallas as pl
from jax.experimental.pallas import tpu as pltpu
from jax.experimental.pallas import tpu_sc as plsc

KNN = 20
KPAD = 24
RT = 256
CHUNK = 128


def _topk_body(cc_ref, ct_ref, out_ref):
    n = cc_ref.shape[1]
    r = ct_ref.shape[2]
    cc = cc_ref[0]
    ct = ct_ref[0]
    iota_f = lax.broadcasted_iota(jnp.int32, (n, r), 0).astype(jnp.float32)
    col_f = (lax.broadcasted_iota(jnp.int32, (n, r), 1)
             + pl.program_id(1) * r).astype(jnp.float32)
    d = ((cc[:, 0:1] - ct[0:1, :]) ** 2
         + (cc[:, 1:2] - ct[1:2, :]) ** 2)
    d = jnp.where(iota_f == col_f, jnp.inf, d)
    base = pl.program_id(0) * n
    for j in range(KNN):
        m = jnp.min(d, axis=0, keepdims=True)
        msk = d == m
        a_f = jnp.min(jnp.where(msk, iota_f, 4096.0), axis=0)
        out_ref[0, j, :] = a_f.astype(jnp.int32) + base
        if j < KNN - 1:
            d = jnp.where(msk, jnp.inf, d)
    out_ref[0, KNN:KPAD, :] = jnp.zeros((KPAD - KNN, r), jnp.int32)


def _mlp_body(x_ref, nb_ref, w1_ref, b1_ref, w2_ref, b2_ref, out_ref):
    d = x_ref.shape[2]
    r = x_ref.shape[1]
    w1a = w1_ref[0:d, :]
    w1b = w1_ref[d:2 * d, :]
    u = jnp.dot(x_ref[0], w1a - w1b,
                preferred_element_type=jnp.float32) + b1_ref[...]
    nb = nb_ref[0].reshape(KNN * r, d)
    g = jnp.dot(nb, w1b, preferred_element_type=jnp.float32)
    units = g.shape[1]
    h1 = jnp.maximum(g.reshape(KNN, r, units) + u[None], 0.0)
    h2 = jnp.maximum(
        jnp.dot(h1.reshape(KNN * r, units), w2_ref[...],
                preferred_element_type=jnp.float32) + b2_ref[...], 0.0)
    out_ref[0] = jnp.max(h2.reshape(KNN, r, units), axis=0)


def _sc_gather(xf, idx2, b_sz, n, d):
    info = plsc.get_sparse_core_info()
    nc, ns = info.num_cores, info.num_subcores
    nw = nc * ns
    n_tasks = b_sz * KNN
    tasks_per_w = -(-n_tasks // nw)
    chunks = n // CHUNK
    mesh = plsc.VectorSubcoreMesh(core_axis_name="c", subcore_axis_name="s")

    @functools.partial(
        pl.kernel, mesh=mesh,
        out_type=jax.ShapeDtypeStruct((b_sz, KNN, n, d), jnp.float32),
        scratch_types=[
            pltpu.VMEM((chunks, CHUNK), jnp.int32),
            pltpu.VMEM((CHUNK, d), jnp.float32),
            pltpu.SemaphoreType.DMA,
        ],
        compiler_params=pltpu.CompilerParams(use_tc_tiling_on_sc=False),
    )
    def gather_kernel(xf_hbm, idx_hbm, out_hbm, idx_v, rows_v, sem):
        wid = lax.axis_index("s") * nc + lax.axis_index("c")
        for t in range(tasks_per_w):
            tau = wid * tasks_per_w + t

            @pl.when(tau < n_tasks)
            def _run():
                b = tau // KNN
                k = tau - b * KNN
                irow = (b * KPAD + k) * chunks
                pltpu.sync_copy(idx_hbm.at[pl.ds(irow, chunks)], idx_v)

                def chunk_body(c, carry):
                    pltpu.async_copy(xf_hbm.at[idx_v.at[c]], rows_v,
                                     sem).wait()
                    pltpu.sync_copy(rows_v,
                                    out_hbm.at[b, k, pl.ds(c * CHUNK, CHUNK)])
                    return carry

                lax.fori_loop(0, chunks, chunk_body, 0)

    return gather_kernel(xf, idx2)


def kernel(x, W1, b1, W2, b2):
    B, N, D = x.shape
    units = W2.shape[0]
    coords = x[:, :, 1:3]
    cc = jnp.pad(coords, ((0, 0), (0, 0), (0, 6)))
    ct = jnp.pad(jnp.swapaxes(coords, 1, 2), ((0, 0), (0, 6), (0, 0)))

    idxg = pl.pallas_call(
        _topk_body,
        grid=(B, N // RT),
        in_specs=[
            pl.BlockSpec((1, N, 8), lambda b, t: (b, 0, 0)),
            pl.BlockSpec((1, 8, RT), lambda b, t: (b, 0, t)),
        ],
        out_specs=pl.BlockSpec((1, KPAD, RT), lambda b, t: (b, 0, t)),
        out_shape=jax.ShapeDtypeStruct((B, KPAD, N), jnp.int32),
    )(cc, ct)

    idx2 = idxg.reshape(B * KPAD * (N // CHUNK), CHUNK)
    xf = x.reshape(B * N, D)
    rows_per_b = KPAD * (N // CHUNK)

    mlp = pl.pallas_call(
        _mlp_body,
        grid=(1, N // RT),
        in_specs=[
            pl.BlockSpec((1, RT, D), lambda b, t: (b, t, 0)),
            pl.BlockSpec((1, KNN, RT, D), lambda b, t: (b, 0, t, 0)),
            pl.BlockSpec((2 * D, units), lambda b, t: (0, 0)),
            pl.BlockSpec((1, units), lambda b, t: (0, 0)),
            pl.BlockSpec((units, units), lambda b, t: (0, 0)),
            pl.BlockSpec((1, units), lambda b, t: (0, 0)),
        ],
        out_specs=pl.BlockSpec((1, RT, units), lambda b, t: (b, t, 0)),
        out_shape=jax.ShapeDtypeStruct((1, N, units), jnp.float32),
    )
    b1r = b1.reshape(1, units)
    b2r = b2.reshape(1, units)
    outs = []
    for b in range(B):
        idx2_b = lax.slice_in_dim(idx2, b * rows_per_b, (b + 1) * rows_per_b)
        nb_b = _sc_gather(xf, idx2_b, 1, N, D)
        outs.append(mlp(x[b:b + 1], nb_b, W1, b1r, W2, b2r))
    return jnp.concatenate(outs, axis=0)


# speedup vs baseline: 1.0585x; 1.0357x over previous
import functools

import jax
import jax.numpy as jnp
from jax import lax
from jax.experimental import pallas as pl
from jax.experimental.pallas import tpu as pltpu
from jax.experimental.pallas import tpu_sc as plsc

KNN = 20
KPAD = 24
RT = 256
CHUNK = 128


def _topk_body(cc_ref, ct_ref, out_ref):
    n = cc_ref.shape[1]
    r = ct_ref.shape[2]
    cc = cc_ref[0]
    ct = ct_ref[0]
    iota_f = lax.broadcasted_iota(jnp.int32, (n, r), 0).astype(jnp.float32)
    col_f = (lax.broadcasted_iota(jnp.int32, (n, r), 1)
             + pl.program_id(1) * r).astype(jnp.float32)
    d = ((cc[:, 0:1] - ct[0:1, :]) ** 2
         + (cc[:, 1:2] - ct[1:2, :]) ** 2)
    d = jnp.where(iota_f == col_f, jnp.inf, d)
    base = pl.program_id(0) * n
    for j in range(KNN):
        m = jnp.min(d, axis=0, keepdims=True)
        msk = d == m
        a_f = jnp.min(jnp.where(msk, iota_f, 4096.0), axis=0)
        out_ref[0, j, :] = a_f.astype(jnp.int32) + base
        if j < KNN - 1:
            d = jnp.where(msk, jnp.inf, d)
    out_ref[0, KNN:KPAD, :] = jnp.zeros((KPAD - KNN, r), jnp.int32)


def _mlp_body(x_ref, nb_ref, w1_ref, b1_ref, w2_ref, b2_ref, out_ref):
    d = x_ref.shape[2]
    r = x_ref.shape[1]
    w1a = w1_ref[0:d, :]
    w1b = w1_ref[d:2 * d, :]
    u = jnp.dot(x_ref[0], w1a - w1b,
                preferred_element_type=jnp.float32) + b1_ref[...]
    nb = nb_ref[0].reshape(KNN * r, d)
    g = jnp.dot(nb, w1b, preferred_element_type=jnp.float32)
    units = g.shape[1]
    h1 = jnp.maximum(g.reshape(KNN, r, units) + u[None], 0.0)
    h2 = jnp.maximum(
        jnp.dot(h1.reshape(KNN * r, units), w2_ref[...],
                preferred_element_type=jnp.float32) + b2_ref[...], 0.0)
    out_ref[0] = jnp.max(h2.reshape(KNN, r, units), axis=0)


def _sc_gather(xf, idx2, b_sz, n, d):
    info = plsc.get_sparse_core_info()
    nc, ns = info.num_cores, info.num_subcores
    nw = nc * ns
    n_tasks = b_sz * KNN
    tasks_per_w = n_tasks // nw
    chunks = n // CHUNK
    mesh = plsc.VectorSubcoreMesh(core_axis_name="c", subcore_axis_name="s")

    @functools.partial(
        pl.kernel, mesh=mesh,
        out_type=jax.ShapeDtypeStruct((b_sz, KNN, n, d), jnp.float32),
        scratch_types=[
            pltpu.VMEM((chunks, CHUNK), jnp.int32),
            pltpu.VMEM((CHUNK, d), jnp.float32),
            pltpu.SemaphoreType.DMA,
        ],
        compiler_params=pltpu.CompilerParams(use_tc_tiling_on_sc=False),
    )
    def gather_kernel(xf_hbm, idx_hbm, out_hbm, idx_v, rows_v, sem):
        wid = lax.axis_index("s") * nc + lax.axis_index("c")
        for t in range(tasks_per_w):
            tau = wid * tasks_per_w + t
            b = tau // KNN
            k = tau - b * KNN
            irow = (b * KPAD + k) * chunks
            pltpu.sync_copy(idx_hbm.at[pl.ds(irow, chunks)], idx_v)

            def chunk_body(c, carry):
                pltpu.async_copy(xf_hbm.at[idx_v.at[c]], rows_v, sem).wait()
                pltpu.sync_copy(rows_v,
                                out_hbm.at[b, k, pl.ds(c * CHUNK, CHUNK)])
                return carry

            lax.fori_loop(0, chunks, chunk_body, 0)

    return gather_kernel(xf, idx2)


def kernel(x, W1, b1, W2, b2):
    B, N, D = x.shape
    units = W2.shape[0]
    coords = x[:, :, 1:3]
    cc = jnp.pad(coords, ((0, 0), (0, 0), (0, 6)))
    ct = jnp.pad(jnp.swapaxes(coords, 1, 2), ((0, 0), (0, 6), (0, 0)))

    idxg = pl.pallas_call(
        _topk_body,
        grid=(B, N // RT),
        in_specs=[
            pl.BlockSpec((1, N, 8), lambda b, t: (b, 0, 0)),
            pl.BlockSpec((1, 8, RT), lambda b, t: (b, 0, t)),
        ],
        out_specs=pl.BlockSpec((1, KPAD, RT), lambda b, t: (b, 0, t)),
        out_shape=jax.ShapeDtypeStruct((B, KPAD, N), jnp.int32),
        compiler_params=pltpu.CompilerParams(
            dimension_semantics=("parallel", "parallel")),
    )(cc, ct)

    idx2 = idxg.reshape(B * KPAD * (N // CHUNK), CHUNK)
    xf = x.reshape(B * N, D)
    nb4 = _sc_gather(xf, idx2, B, N, D)

    out = pl.pallas_call(
        _mlp_body,
        grid=(B, N // RT),
        in_specs=[
            pl.BlockSpec((1, RT, D), lambda b, t: (b, t, 0)),
            pl.BlockSpec((1, KNN, RT, D), lambda b, t: (b, 0, t, 0)),
            pl.BlockSpec((2 * D, units), lambda b, t: (0, 0)),
            pl.BlockSpec((1, units), lambda b, t: (0, 0)),
            pl.BlockSpec((units, units), lambda b, t: (0, 0)),
            pl.BlockSpec((1, units), lambda b, t: (0, 0)),
        ],
        out_specs=pl.BlockSpec((1, RT, units), lambda b, t: (b, t, 0)),
        out_shape=jax.ShapeDtypeStruct((B, N, units), jnp.float32),
        compiler_params=pltpu.CompilerParams(
            dimension_semantics=("parallel", "parallel")),
    )(x, nb4, W1, b1.reshape(1, units), W2, b2.reshape(1, units))
    return out


# speedup vs baseline: 1.1011x; 1.0402x over previous
import functools

import jax
import jax.numpy as jnp
from jax import lax
from jax.experimental import pallas as pl
from jax.experimental.pallas import tpu as pltpu
from jax.experimental.pallas import tpu_sc as plsc

KNN = 20
KPAD = 24
RT = 256
RTK = 512
CHUNK = 128


def _topk_body(cc_ref, ct_ref, out_ref):
    n = cc_ref.shape[1]
    r = ct_ref.shape[2]
    cc = cc_ref[0]
    ct = ct_ref[0]
    iota_f = lax.broadcasted_iota(jnp.int32, (n, r), 0).astype(jnp.float32)
    col_f = (lax.broadcasted_iota(jnp.int32, (n, r), 1)
             + pl.program_id(1) * r).astype(jnp.float32)
    d = ((cc[:, 0:1] - ct[0:1, :]) ** 2
         + (cc[:, 1:2] - ct[1:2, :]) ** 2)
    d = jnp.where(iota_f == col_f, jnp.inf, d)
    base = pl.program_id(0) * n
    for j in range(KNN):
        m = jnp.min(d, axis=0, keepdims=True)
        msk = d == m
        a_f = jnp.min(jnp.where(msk, iota_f, 4096.0), axis=0)
        out_ref[0, j, :] = a_f.astype(jnp.int32) + base
        if j < KNN - 1:
            d = jnp.where(msk, jnp.inf, d)
    out_ref[0, KNN:KPAD, :] = jnp.zeros((KPAD - KNN, r), jnp.int32)


def _mlp_body(x_ref, nb_ref, w1_ref, b1_ref, w2_ref, b2_ref, out_ref):
    d = x_ref.shape[2]
    r = x_ref.shape[1]
    w1a = w1_ref[0:d, :]
    w1b = w1_ref[d:2 * d, :]
    u = jnp.dot(x_ref[0], w1a - w1b,
                preferred_element_type=jnp.float32) + b1_ref[...]
    nb = nb_ref[0].reshape(KNN * r, d)
    g = jnp.dot(nb, w1b, preferred_element_type=jnp.float32)
    units = g.shape[1]
    h1 = jnp.maximum(g.reshape(KNN, r, units) + u[None], 0.0)
    h2 = jnp.maximum(
        jnp.dot(h1.reshape(KNN * r, units), w2_ref[...],
                preferred_element_type=jnp.float32) + b2_ref[...], 0.0)
    out_ref[0] = jnp.max(h2.reshape(KNN, r, units), axis=0)


def _sc_gather(xf, idx2, b_sz, n, d):
    info = plsc.get_sparse_core_info()
    nc, ns = info.num_cores, info.num_subcores
    nw = nc * ns
    n_tasks = b_sz * KNN
    tasks_per_w = n_tasks // nw
    chunks = n // CHUNK
    mesh = plsc.VectorSubcoreMesh(core_axis_name="c", subcore_axis_name="s")

    @functools.partial(
        pl.kernel, mesh=mesh,
        out_type=jax.ShapeDtypeStruct((b_sz, KNN, n, d), jnp.float32),
        scratch_types=[
            pltpu.VMEM((chunks, CHUNK), jnp.int32),
            pltpu.VMEM((CHUNK, d), jnp.float32),
            pltpu.SemaphoreType.DMA,
        ],
        compiler_params=pltpu.CompilerParams(use_tc_tiling_on_sc=False),
    )
    def gather_kernel(xf_hbm, idx_hbm, out_hbm, idx_v, rows_v, sem):
        wid = lax.axis_index("s") * nc + lax.axis_index("c")
        for t in range(tasks_per_w):
            tau = wid * tasks_per_w + t
            b = tau // KNN
            k = tau - b * KNN
            irow = (b * KPAD + k) * chunks
            pltpu.sync_copy(idx_hbm.at[pl.ds(irow, chunks)], idx_v)

            def chunk_body(c, carry):
                pltpu.async_copy(xf_hbm.at[idx_v.at[c]], rows_v, sem).wait()
                pltpu.sync_copy(rows_v,
                                out_hbm.at[b, k, pl.ds(c * CHUNK, CHUNK)])
                return carry

            lax.fori_loop(0, chunks, chunk_body, 0)

    return gather_kernel(xf, idx2)


def kernel(x, W1, b1, W2, b2):
    B, N, D = x.shape
    units = W2.shape[0]
    coords = x[:, :, 1:3]
    cc = jnp.pad(coords, ((0, 0), (0, 0), (0, 6)))
    ct = jnp.pad(jnp.swapaxes(coords, 1, 2), ((0, 0), (0, 6), (0, 0)))

    idxg = pl.pallas_call(
        _topk_body,
        grid=(B, N // RTK),
        in_specs=[
            pl.BlockSpec((1, N, 8), lambda b, t: (b, 0, 0)),
            pl.BlockSpec((1, 8, RTK), lambda b, t: (b, 0, t)),
        ],
        out_specs=pl.BlockSpec((1, KPAD, RTK), lambda b, t: (b, 0, t)),
        out_shape=jax.ShapeDtypeStruct((B, KPAD, N), jnp.int32),
        compiler_params=pltpu.CompilerParams(
            dimension_semantics=("parallel", "parallel")),
    )(cc, ct)

    idx2 = idxg.reshape(B * KPAD * (N // CHUNK), CHUNK)
    xf = x.reshape(B * N, D)
    nb4 = _sc_gather(xf, idx2, B, N, D)

    out = pl.pallas_call(
        _mlp_body,
        grid=(B, N // RT),
        in_specs=[
            pl.BlockSpec((1, RT, D), lambda b, t: (b, t, 0)),
            pl.BlockSpec((1, KNN, RT, D), lambda b, t: (b, 0, t, 0)),
            pl.BlockSpec((2 * D, units), lambda b, t: (0, 0)),
            pl.BlockSpec((1, units), lambda b, t: (0, 0)),
            pl.BlockSpec((units, units), lambda b, t: (0, 0)),
            pl.BlockSpec((1, units), lambda b, t: (0, 0)),
        ],
        out_specs=pl.BlockSpec((1, RT, units), lambda b, t: (b, t, 0)),
        out_shape=jax.ShapeDtypeStruct((B, N, units), jnp.float32),
        compiler_params=pltpu.CompilerParams(
            dimension_semantics=("parallel", "parallel")),
    )(x, nb4, W1, b1.reshape(1, units), W2, b2.reshape(1, units))
    return out
